# baseline (speedup 1.0000x reference)
import functools

import jax
import jax.numpy as jnp
from jax import lax
from jax.experimental import pallas as pl
from jax.experimental.pallas import tpu as pltpu
from jax.experimental.pallas import tpu_sc as plsc

NODE_NUM = 100000
N_UPDATES = 1600000

NUM_CORES = 2
NUM_SUBCORES = 16
NUM_WORKERS = NUM_CORES * NUM_SUBCORES
PER_WORKER = N_UPDATES // NUM_WORKERS
CHUNK = 10000
NUM_CHUNKS = PER_WORKER // CHUNK
NODE_PAD = 100096
SLICE = NODE_PAD // NUM_SUBCORES


def _sc_accumulate(idx_hbm, val_hbm, zeros_hbm, ones_hbm,
                   sums_out, counts_out,
                   idx_v0, idx_v1, val_v0, val_v1, ones_v, stage_v,
                   sums_sh, counts_sh,
                   isem0, isem1, vsem0, vsem1):
    cid = lax.axis_index("c")
    sid = lax.axis_index("s")
    wid = cid * NUM_SUBCORES + sid
    base = wid * PER_WORKER
    idx_v = (idx_v0, idx_v1)
    val_v = (val_v0, val_v1)
    isem = (isem0, isem1)
    vsem = (vsem0, vsem1)

    def load(k):
        off = base + k * CHUNK
        b = k % 2
        return (
            pltpu.async_copy(idx_hbm.at[pl.ds(off, CHUNK)], idx_v[b], isem[b]),
            pltpu.async_copy(val_hbm.at[pl.ds(off, CHUNK)], val_v[b], vsem[b]),
        )

    lds = [load(0), load(1)]
    pltpu.sync_copy(zeros_hbm, stage_v)
    pltpu.sync_copy(stage_v, sums_sh.at[pl.ds(sid * SLICE, SLICE)])
    pltpu.sync_copy(stage_v, counts_sh.at[pl.ds(sid * SLICE, SLICE)])
    pltpu.sync_copy(ones_hbm, ones_v)
    plsc.subcore_barrier()

    for k in range(NUM_CHUNKS):
        b = k % 2
        for c in lds[k]:
            c.wait()
        pltpu.sync_copy(val_v[b], sums_sh.at[idx_v[b]], add=True)
        pltpu.sync_copy(ones_v, counts_sh.at[idx_v[b]], add=True)
        if k + 2 < NUM_CHUNKS:
            lds.append(load(k + 2))
    plsc.subcore_barrier()

    sl = pl.ds(sid * SLICE, SLICE)
    osl = pl.ds(cid * NODE_PAD + sid * SLICE, SLICE)
    pltpu.sync_copy(sums_sh.at[sl], stage_v)
    pltpu.sync_copy(stage_v, sums_out.at[osl])
    pltpu.sync_copy(counts_sh.at[sl], stage_v)
    pltpu.sync_copy(stage_v, counts_out.at[osl])


_sc_call = functools.partial(
    pl.kernel,
    out_type=(
        jax.ShapeDtypeStruct((NUM_CORES * NODE_PAD,), jnp.float32),
        jax.ShapeDtypeStruct((NUM_CORES * NODE_PAD,), jnp.float32),
    ),
    mesh=plsc.VectorSubcoreMesh(core_axis_name="c", subcore_axis_name="s"),
    scratch_types=(
        pltpu.VMEM((CHUNK,), jnp.int32),
        pltpu.VMEM((CHUNK,), jnp.int32),
        pltpu.VMEM((CHUNK,), jnp.float32),
        pltpu.VMEM((CHUNK,), jnp.float32),
        pltpu.VMEM((CHUNK,), jnp.float32),
        pltpu.VMEM((SLICE,), jnp.float32),
        pltpu.VMEM_SHARED((NODE_PAD,), jnp.float32),
        pltpu.VMEM_SHARED((NODE_PAD,), jnp.float32),
        pltpu.SemaphoreType.DMA,
        pltpu.SemaphoreType.DMA,
        pltpu.SemaphoreType.DMA,
        pltpu.SemaphoreType.DMA,
    ),
)(_sc_accumulate)


def _combine_body(state_ref, sums_ref, counts_ref, out_ref):
    s = sums_ref[pl.ds(0, NODE_NUM)] + sums_ref[pl.ds(NODE_PAD, NODE_NUM)]
    c = counts_ref[pl.ds(0, NODE_NUM)] + counts_ref[pl.ds(NODE_PAD, NODE_NUM)]
    out_ref[...] = state_ref[...] + s / jnp.maximum(c, 1.0)


def kernel(node_errors_state, node_errors, node_indices):
    zeros = jnp.zeros((SLICE,), jnp.float32)
    ones = jnp.ones((CHUNK,), jnp.float32)
    sums, counts = _sc_call(node_indices, node_errors, zeros, ones)
    return pl.pallas_call(
        _combine_body,
        out_shape=jax.ShapeDtypeStruct((NODE_NUM,), jnp.float32),
    )(node_errors_state, sums, counts)


# speedup vs baseline: 1.0212x; 1.0212x over previous
import functools

import jax
import jax.numpy as jnp
from jax import lax
from jax.experimental import pallas as pl
from jax.experimental.pallas import tpu as pltpu
from jax.experimental.pallas import tpu_sc as plsc

NODE_NUM = 100000
N_UPDATES = 1600000

NUM_CORES = 2
NUM_SUBCORES = 16
NUM_WORKERS = NUM_CORES * NUM_SUBCORES
PER_WORKER = N_UPDATES // NUM_WORKERS
CHUNK = 10000
NUM_CHUNKS = PER_WORKER // CHUNK
NODE_PAD = 100096
SLICE = NODE_PAD // NUM_SUBCORES


def _sc_accumulate(idx_hbm, val_hbm,
                   sums_out, counts_out,
                   idx_v0, idx_v1, val_v0, val_v1, ones_v, stage_v,
                   sums_sh, counts_sh,
                   isem0, isem1, vsem0, vsem1, csem):
    cid = lax.axis_index("c")
    sid = lax.axis_index("s")
    wid = cid * NUM_SUBCORES + sid
    base = wid * PER_WORKER
    idx_v = (idx_v0, idx_v1)
    val_v = (val_v0, val_v1)
    isem = (isem0, isem1)
    vsem = (vsem0, vsem1)

    def load(k):
        off = base + k * CHUNK
        b = k % 2
        return (
            pltpu.async_copy(idx_hbm.at[pl.ds(off, CHUNK)], idx_v[b], isem[b]),
            pltpu.async_copy(val_hbm.at[pl.ds(off, CHUNK)], val_v[b], vsem[b]),
        )

    lds = [load(0), load(1)]

    def fill_zero(i, carry):
        stage_v[pl.ds(i * 16, 16)] = jnp.zeros((16,), jnp.float32)
        return carry

    def fill_one(i, carry):
        ones_v[pl.ds(i * 16, 16)] = jnp.ones((16,), jnp.float32)
        return carry

    lax.fori_loop(0, SLICE // 16, fill_zero, 0)
    lax.fori_loop(0, CHUNK // 16, fill_one, 0)
    pltpu.sync_copy(stage_v, sums_sh.at[pl.ds(sid * SLICE, SLICE)])
    pltpu.sync_copy(stage_v, counts_sh.at[pl.ds(sid * SLICE, SLICE)])
    plsc.subcore_barrier()

    for k in range(NUM_CHUNKS):
        b = k % 2
        for c in lds[k]:
            c.wait()
        cnt = pltpu.async_copy(ones_v, counts_sh.at[idx_v[b]], csem, add=True)
        pltpu.sync_copy(val_v[b], sums_sh.at[idx_v[b]], add=True)
        cnt.wait()
        if k + 2 < NUM_CHUNKS:
            lds.append(load(k + 2))
    plsc.subcore_barrier()

    sl = pl.ds(sid * SLICE, SLICE)
    osl = pl.ds(cid * NODE_PAD + sid * SLICE, SLICE)
    pltpu.sync_copy(sums_sh.at[sl], stage_v)
    pltpu.sync_copy(stage_v, sums_out.at[osl])
    pltpu.sync_copy(counts_sh.at[sl], stage_v)
    pltpu.sync_copy(stage_v, counts_out.at[osl])


_sc_call = functools.partial(
    pl.kernel,
    out_type=(
        jax.ShapeDtypeStruct((NUM_CORES * NODE_PAD,), jnp.float32),
        jax.ShapeDtypeStruct((NUM_CORES * NODE_PAD,), jnp.float32),
    ),
    mesh=plsc.VectorSubcoreMesh(core_axis_name="c", subcore_axis_name="s"),
    scratch_types=(
        pltpu.VMEM((CHUNK,), jnp.int32),
        pltpu.VMEM((CHUNK,), jnp.int32),
        pltpu.VMEM((CHUNK,), jnp.float32),
        pltpu.VMEM((CHUNK,), jnp.float32),
        pltpu.VMEM((CHUNK,), jnp.float32),
        pltpu.VMEM((SLICE,), jnp.float32),
        pltpu.VMEM_SHARED((NODE_PAD,), jnp.float32),
        pltpu.VMEM_SHARED((NODE_PAD,), jnp.float32),
        pltpu.SemaphoreType.DMA,
        pltpu.SemaphoreType.DMA,
        pltpu.SemaphoreType.DMA,
        pltpu.SemaphoreType.DMA,
        pltpu.SemaphoreType.DMA,
    ),
)(_sc_accumulate)


def _combine_body(state_ref, sums_ref, counts_ref, out_ref):
    s = sums_ref[pl.ds(0, NODE_NUM)] + sums_ref[pl.ds(NODE_PAD, NODE_NUM)]
    c = counts_ref[pl.ds(0, NODE_NUM)] + counts_ref[pl.ds(NODE_PAD, NODE_NUM)]
    out_ref[...] = state_ref[...] + s / jnp.maximum(c, 1.0)


def kernel(node_errors_state, node_errors, node_indices):
    sums, counts = _sc_call(node_indices, node_errors)
    return pl.pallas_call(
        _combine_body,
        out_shape=jax.ShapeDtypeStruct((NODE_NUM,), jnp.float32),
    )(node_errors_state, sums, counts)


# speedup vs baseline: 1.0346x; 1.0131x over previous
import functools

import jax
import jax.numpy as jnp
from jax import lax
from jax.experimental import pallas as pl
from jax.experimental.pallas import tpu as pltpu
from jax.experimental.pallas import tpu_sc as plsc

NODE_NUM = 100000
N_UPDATES = 1600000

NUM_CORES = 2
NUM_SUBCORES = 16
NUM_WORKERS = NUM_CORES * NUM_SUBCORES
PER_WORKER = N_UPDATES // NUM_WORKERS
CHUNK = 10000
NUM_CHUNKS = PER_WORKER // CHUNK
NODE_PAD = 100096
SLICE = NODE_PAD // NUM_SUBCORES


def _sc_accumulate(idx_hbm, val_hbm,
                   sums_out, counts_out,
                   idx_v0, idx_v1, val_v0, val_v1, ones_v, stage_v,
                   sums_sh, counts_sh,
                   isem0, isem1, vsem0, vsem1, zsem0, zsem1):
    cid = lax.axis_index("c")
    sid = lax.axis_index("s")
    wid = cid * NUM_SUBCORES + sid
    base = wid * PER_WORKER
    idx_v = (idx_v0, idx_v1)
    val_v = (val_v0, val_v1)
    isem = (isem0, isem1)
    vsem = (vsem0, vsem1)

    def load(k):
        off = base + k * CHUNK
        b = k % 2
        return (
            pltpu.async_copy(idx_hbm.at[pl.ds(off, CHUNK)], idx_v[b], isem[b]),
            pltpu.async_copy(val_hbm.at[pl.ds(off, CHUNK)], val_v[b], vsem[b]),
        )

    lds = [load(0), load(1)]

    def fill_zero(i, carry):
        stage_v[pl.ds(i * 16, 16)] = jnp.zeros((16,), jnp.float32)
        return carry

    def fill_one(i, carry):
        ones_v[pl.ds(i * 16, 16)] = jnp.ones((16,), jnp.float32)
        return carry

    lax.fori_loop(0, SLICE // 16, fill_zero, 0)
    z1 = pltpu.async_copy(stage_v, sums_sh.at[pl.ds(sid * SLICE, SLICE)],
                          zsem0)
    z2 = pltpu.async_copy(stage_v, counts_sh.at[pl.ds(sid * SLICE, SLICE)],
                          zsem1)
    lax.fori_loop(0, CHUNK // 16, fill_one, 0)
    z1.wait()
    z2.wait()
    plsc.subcore_barrier()

    for k in range(NUM_CHUNKS):
        b = k % 2
        for c in lds[k]:
            c.wait()
        pltpu.sync_copy(val_v[b], sums_sh.at[idx_v[b]], add=True)
        pltpu.sync_copy(ones_v, counts_sh.at[idx_v[b]], add=True)
        if k + 2 < NUM_CHUNKS:
            lds.append(load(k + 2))
    plsc.subcore_barrier()

    sl = pl.ds(sid * SLICE, SLICE)
    osl = pl.ds(cid * NODE_PAD + sid * SLICE, SLICE)
    stage_c = ones_v.at[pl.ds(0, SLICE)]
    d1 = pltpu.async_copy(sums_sh.at[sl], stage_v, zsem0)
    d2 = pltpu.async_copy(counts_sh.at[sl], stage_c, zsem1)
    d1.wait()
    d2.wait()
    d3 = pltpu.async_copy(stage_v, sums_out.at[osl], zsem0)
    d4 = pltpu.async_copy(stage_c, counts_out.at[osl], zsem1)
    d3.wait()
    d4.wait()


_sc_call = functools.partial(
    pl.kernel,
    out_type=(
        jax.ShapeDtypeStruct((NUM_CORES * NODE_PAD,), jnp.float32),
        jax.ShapeDtypeStruct((NUM_CORES * NODE_PAD,), jnp.float32),
    ),
    mesh=plsc.VectorSubcoreMesh(core_axis_name="c", subcore_axis_name="s"),
    scratch_types=(
        pltpu.VMEM((CHUNK,), jnp.int32),
        pltpu.VMEM((CHUNK,), jnp.int32),
        pltpu.VMEM((CHUNK,), jnp.float32),
        pltpu.VMEM((CHUNK,), jnp.float32),
        pltpu.VMEM((CHUNK,), jnp.float32),
        pltpu.VMEM((SLICE,), jnp.float32),
        pltpu.VMEM_SHARED((NODE_PAD,), jnp.float32),
        pltpu.VMEM_SHARED((NODE_PAD,), jnp.float32),
        pltpu.SemaphoreType.DMA,
        pltpu.SemaphoreType.DMA,
        pltpu.SemaphoreType.DMA,
        pltpu.SemaphoreType.DMA,
        pltpu.SemaphoreType.DMA,
        pltpu.SemaphoreType.DMA,
    ),
)(_sc_accumulate)


def _combine_body(state_ref, sums_ref, counts_ref, out_ref):
    s = sums_ref[pl.ds(0, NODE_NUM)] + sums_ref[pl.ds(NODE_PAD, NODE_NUM)]
    c = counts_ref[pl.ds(0, NODE_NUM)] + counts_ref[pl.ds(NODE_PAD, NODE_NUM)]
    out_ref[...] = state_ref[...] + s / jnp.maximum(c, 1.0)


def kernel(node_errors_state, node_errors, node_indices):
    sums, counts = _sc_call(node_indices, node_errors)
    return pl.pallas_call(
        _combine_body,
        out_shape=jax.ShapeDtypeStruct((NODE_NUM,), jnp.float32),
    )(node_errors_state, sums, counts)
